# baseline (speedup 1.0000x reference)
import functools

import jax
import jax.numpy as jnp
from jax import lax
from jax.experimental import pallas as pl
from jax.experimental.pallas import tpu as pltpu
from jax.experimental.pallas import tpu_sc as plsc

_N, _E, _D, _H, _C = 10000, 160000, 256, 512, 16
_F1, _F2 = 32, 16
_NSC = 2
_NSUB = 16
_NW = _NSC * _NSUB
_CH = 1024
_NB = _CH // 128
_EPAD = -(-_E // (_NW * _CH)) * (_NW * _CH)
_EPW = _EPAD // _NW
_NCHUNK = _EPW // _CH
_BR = 2000
_RPT = _N // _NSUB


def _invsum_body(w_ref, o_ref):
    o_ref[...] = (1.0 / jnp.sum(w_ref[...]))[None, None]


_invsum = pl.pallas_call(
    _invsum_body,
    out_shape=jax.ShapeDtypeStruct((1, 1), jnp.float32),
)


def _pre_body(inv_ref, x_ref, w1_ref, b1_ref, w2_ref, b2_ref, wg_ref, g_ref):
    h = jax.nn.gelu(
        jnp.dot(x_ref[...], w1_ref[...], preferred_element_type=jnp.float32)
        + b1_ref[...])
    h = jax.nn.gelu(
        jnp.dot(h.astype(jnp.bfloat16), w2_ref[...],
                preferred_element_type=jnp.float32)
        + b2_ref[...])
    g_ref[...] = jnp.dot(
        h.astype(jnp.bfloat16), wg_ref[...],
        preferred_element_type=jnp.float32) * inv_ref[...]


_pre = pl.pallas_call(
    _pre_body,
    grid=(_N // _BR,),
    in_specs=[
        pl.BlockSpec((1, 1), lambda i: (0, 0)),
        pl.BlockSpec((_BR, _D), lambda i: (i, 0)),
        pl.BlockSpec((_D, _H), lambda i: (0, 0)),
        pl.BlockSpec((1, _H), lambda i: (0, 0)),
        pl.BlockSpec((_H, _H), lambda i: (0, 0)),
        pl.BlockSpec((1, _H), lambda i: (0, 0)),
        pl.BlockSpec((_H, _F1), lambda i: (0, 0)),
    ],
    out_specs=pl.BlockSpec((_BR, _F1), lambda i: (i, 0)),
    out_shape=jax.ShapeDtypeStruct((_N, _F1), jnp.float32),
)


def _post_body(p_ref, b_ref, w1_ref, b1_ref, w2_ref, b2_ref, wo_ref, bo_ref,
               y_ref):
    h = jnp.maximum(p_ref[0] + p_ref[1] + b_ref[...], 0.0)
    h = jax.nn.gelu(
        jnp.dot(h.astype(jnp.bfloat16), w1_ref[...],
                preferred_element_type=jnp.float32)
        + b1_ref[...])
    h = jax.nn.gelu(
        jnp.dot(h.astype(jnp.bfloat16), w2_ref[...],
                preferred_element_type=jnp.float32)
        + b2_ref[...])
    y_ref[...] = (
        jnp.dot(h.astype(jnp.bfloat16), wo_ref[...],
                preferred_element_type=jnp.float32)
        + bo_ref[...])


def _mid_body(inv_ref, p_ref, b_ref, wg_ref, g_ref):
    h = jnp.maximum(p_ref[0] + p_ref[1] + b_ref[...], 0.0)
    g_ref[...] = jnp.dot(
        h, wg_ref[...], preferred_element_type=jnp.float32) * inv_ref[...]


_mid = pl.pallas_call(
    _mid_body,
    grid=(_N // _BR,),
    in_specs=[
        pl.BlockSpec((1, 1), lambda i: (0, 0)),
        pl.BlockSpec((_NSC, _BR, _F1), lambda i: (0, i, 0)),
        pl.BlockSpec((1, _F1), lambda i: (0, 0)),
        pl.BlockSpec((_F1, _F2), lambda i: (0, 0)),
    ],
    out_specs=pl.BlockSpec((_BR, _F2), lambda i: (i, 0)),
    out_shape=jax.ShapeDtypeStruct((_N, _F2), jnp.float32),
)


_post = pl.pallas_call(
    _post_body,
    grid=(_N // _BR,),
    in_specs=[
        pl.BlockSpec((_NSC, _BR, _F2), lambda i: (0, i, 0)),
        pl.BlockSpec((1, _F2), lambda i: (0, 0)),
        pl.BlockSpec((_F2, _H), lambda i: (0, 0)),
        pl.BlockSpec((1, _H), lambda i: (0, 0)),
        pl.BlockSpec((_H, _H), lambda i: (0, 0)),
        pl.BlockSpec((1, _H), lambda i: (0, 0)),
        pl.BlockSpec((_H, _C), lambda i: (0, 0)),
        pl.BlockSpec((1, _C), lambda i: (0, 0)),
    ],
    out_specs=pl.BlockSpec((_BR, _C), lambda i: (i, 0)),
    out_shape=jax.ShapeDtypeStruct((_N, _C), jnp.float32),
)


def _make_sc_spmm(F):
    mesh = plsc.VectorSubcoreMesh(core_axis_name="c", subcore_axis_name="s")
    nj = F // 16
    dnums = lax.GatherDimensionNumbers(
        offset_dims=(), collapsed_slice_dims=(0,), start_index_map=(0,))

    @functools.partial(
        pl.kernel,
        out_type=jax.ShapeDtypeStruct((_NSC, _N, F), jnp.float32),
        mesh=mesh,
        scratch_types=[
            pltpu.VMEM((_NCHUNK * _NB, 128), jnp.int32),
            pltpu.VMEM((_NCHUNK * _NB, 128), jnp.int32),
            pltpu.VMEM((_EPW,), jnp.float32),
            pltpu.VMEM((2, _CH, F), jnp.float32),
            pltpu.VMEM_SHARED((_N, F), jnp.float32),
            pltpu.SemaphoreType.DMA,
            pltpu.SemaphoreType.DMA,
            pltpu.SemaphoreType.DMA,
            pltpu.SemaphoreType.DMA,
        ],
        compiler_params=pltpu.CompilerParams(use_tc_tiling_on_sc=False),
    )
    def spmm_k(g_hbm, src_hbm, dst_hbm, w_hbm, out_hbm,
               sidx_v, didx_v, wv_v, rows_v, acc,
               gsem0, gsem1, ssem0, ssem1):
        c = lax.axis_index("c")
        s = lax.axis_index("s")
        wid = s * _NSC + c
        gsems = (gsem0, gsem1)
        ssems = (ssem0, ssem1)

        def _fire_gathers(k):
            p = k % 2
            return [
                pltpu.async_copy(
                    g_hbm.at[sidx_v.at[k * _NB + b]],
                    rows_v.at[p, pl.ds(b * 128, 128)], gsems[p])
                for b in range(_NB)
            ]

        nrow = _EPW // 128
        icps = [
            pltpu.async_copy(
                src_hbm.at[pl.ds(wid * nrow, nrow)], sidx_v, gsem1),
            pltpu.async_copy(
                dst_hbm.at[pl.ds(wid * nrow, nrow)], didx_v, gsem1),
            pltpu.async_copy(
                w_hbm.at[pl.ds(wid * _EPW, _EPW)], wv_v, gsem1),
        ]

        def _zbody(i, carry):
            for j in range(nj):
                rows_v[1, i, pl.ds(j * 16, 16)] = jnp.zeros(
                    (16,), jnp.float32)
            return carry
        lax.fori_loop(0, _RPT, _zbody, 0)
        for cp in icps:
            cp.wait()
        gcps = {0: _fire_gathers(0)}
        pltpu.sync_copy(rows_v.at[1, pl.ds(0, _RPT)],
                        acc.at[pl.ds(s * _RPT, _RPT)])
        plsc.subcore_barrier()

        scps = {}
        for k in range(_NCHUNK):
            p = k % 2
            if k + 1 < _NCHUNK:
                if k >= 1:
                    for cp in scps.pop(k - 1):
                        cp.wait()
                gcps[k + 1] = _fire_gathers(k + 1)
            for cp in gcps.pop(k):
                cp.wait()

            def _sbody(t, carry):
                w16 = wv_v[pl.ds(k * _CH + t * 16, 16)]
                for l in range(16):
                    wl = lax.gather(
                        w16, jnp.full((16, 1), l, jnp.int32), dnums, (1,),
                        mode=lax.GatherScatterMode.PROMISE_IN_BOUNDS)
                    e = t * 16 + l
                    for j in range(nj):
                        rows_v[p, e, pl.ds(j * 16, 16)] = (
                            rows_v[p, e, pl.ds(j * 16, 16)] * wl)
                return carry
            lax.fori_loop(0, _CH // 16, _sbody, 0)

            scps[k] = [
                pltpu.async_copy(
                    rows_v.at[p, pl.ds(b * 128, 128)],
                    acc.at[didx_v.at[k * _NB + b]], ssems[p], add=True)
                for b in range(_NB)
            ]
        for k in sorted(scps):
            for cp in scps.pop(k):
                cp.wait()
        plsc.subcore_barrier()

        pltpu.sync_copy(acc.at[pl.ds(s * _RPT, _RPT)],
                        rows_v.at[0, pl.ds(0, _RPT)])
        pltpu.sync_copy(rows_v.at[0, pl.ds(0, _RPT)],
                        out_hbm.at[c, pl.ds(s * _RPT, _RPT)])

    return spmm_k


_spmm1 = _make_sc_spmm(_F1)
_spmm2 = _make_sc_spmm(_F2)


@jax.jit
def kernel(x, edge_index, edge_weight,
           W_pre1, b_pre1, W_pre2, b_pre2,
           W_g1, b_g1, W_g2, b_g2,
           W_post1, b_post1, W_post2, b_post2,
           W_out, b_out):
    pad = _EPAD - _E
    srcp = jnp.pad(edge_index[0], (0, pad)).reshape(_EPAD // 128, 128)
    dstp = jnp.pad(edge_index[1], (0, pad)).reshape(_EPAD // 128, 128)
    wp = jnp.pad(edge_weight, (0, pad))
    inv = _invsum(wp.reshape(-1, 128))

    bf16 = jnp.bfloat16
    g = _pre(inv, x.astype(bf16), W_pre1.astype(bf16), b_pre1.reshape(1, -1),
             W_pre2.astype(bf16), b_pre2.reshape(1, -1), W_g1.astype(bf16))
    p1 = _spmm1(g, srcp, dstp, wp)
    g2 = _mid(inv, p1, b_g1.reshape(1, -1), W_g2)
    p2 = _spmm2(g2, srcp, dstp, wp)
    return _post(p2, b_g2.reshape(1, -1), W_post1.astype(bf16),
                 b_post1.reshape(1, -1), W_post2.astype(bf16),
                 b_post2.reshape(1, -1), W_out.astype(bf16),
                 b_out.reshape(1, -1))


# speedup vs baseline: 1.2915x; 1.2915x over previous
import functools

import jax
import jax.numpy as jnp
from jax import lax
from jax.experimental import pallas as pl
from jax.experimental.pallas import tpu as pltpu
from jax.experimental.pallas import tpu_sc as plsc

_N, _E, _D, _H, _C = 10000, 160000, 256, 512, 16
_F1, _F2 = 32, 16
_NSC = 2
_NSUB = 16
_NW = _NSC * _NSUB
_CH = 1024
_NB = _CH // 128
_EPAD = -(-_E // (_NW * _CH)) * (_NW * _CH)
_EPW = _EPAD // _NW
_NCHUNK = _EPW // _CH
_BR = 2000
_RPT = _N // _NSUB


def _invsum_body(w_ref, o_ref):
    o_ref[...] = (1.0 / jnp.sum(w_ref[...]))[None, None]


_invsum = pl.pallas_call(
    _invsum_body,
    out_shape=jax.ShapeDtypeStruct((1, 1), jnp.float32),
)


def _pre_body(inv_ref, x_ref, w1_ref, b1_ref, w2_ref, b2_ref, wg_ref, g_ref):
    h = jax.nn.gelu(
        jnp.dot(x_ref[...], w1_ref[...], preferred_element_type=jnp.float32)
        + b1_ref[...])
    h = jax.nn.gelu(
        jnp.dot(h.astype(jnp.bfloat16), w2_ref[...],
                preferred_element_type=jnp.float32)
        + b2_ref[...])
    g_ref[...] = jnp.dot(
        h.astype(jnp.bfloat16), wg_ref[...],
        preferred_element_type=jnp.float32) * inv_ref[...]


_pre = pl.pallas_call(
    _pre_body,
    grid=(_N // _BR,),
    in_specs=[
        pl.BlockSpec((1, 1), lambda i: (0, 0)),
        pl.BlockSpec((_BR, _D), lambda i: (i, 0)),
        pl.BlockSpec((_D, _H), lambda i: (0, 0)),
        pl.BlockSpec((1, _H), lambda i: (0, 0)),
        pl.BlockSpec((_H, _H), lambda i: (0, 0)),
        pl.BlockSpec((1, _H), lambda i: (0, 0)),
        pl.BlockSpec((_H, _F1), lambda i: (0, 0)),
    ],
    out_specs=pl.BlockSpec((_BR, _F1), lambda i: (i, 0)),
    out_shape=jax.ShapeDtypeStruct((_N, _F1), jnp.float32),
)


def _post_body(p_ref, b_ref, w1_ref, b1_ref, w2_ref, b2_ref, wo_ref, bo_ref,
               y_ref):
    h = jnp.maximum(p_ref[0] + p_ref[1] + b_ref[...], 0.0)
    h = jax.nn.gelu(
        jnp.dot(h.astype(jnp.bfloat16), w1_ref[...],
                preferred_element_type=jnp.float32)
        + b1_ref[...])
    h = jax.nn.gelu(
        jnp.dot(h.astype(jnp.bfloat16), w2_ref[...],
                preferred_element_type=jnp.float32)
        + b2_ref[...])
    y_ref[...] = (
        jnp.dot(h.astype(jnp.bfloat16), wo_ref[...],
                preferred_element_type=jnp.float32)
        + bo_ref[...])


def _mid_body(inv_ref, p_ref, b_ref, wg_ref, g_ref):
    h = jnp.maximum(p_ref[0] + p_ref[1] + b_ref[...], 0.0)
    g_ref[...] = jnp.dot(
        h, wg_ref[...], preferred_element_type=jnp.float32) * inv_ref[...]


_mid = pl.pallas_call(
    _mid_body,
    grid=(_N // _BR,),
    in_specs=[
        pl.BlockSpec((1, 1), lambda i: (0, 0)),
        pl.BlockSpec((_NSC, _BR, _F1), lambda i: (0, i, 0)),
        pl.BlockSpec((1, _F1), lambda i: (0, 0)),
        pl.BlockSpec((_F1, _F2), lambda i: (0, 0)),
    ],
    out_specs=pl.BlockSpec((_BR, _F2), lambda i: (i, 0)),
    out_shape=jax.ShapeDtypeStruct((_N, _F2), jnp.float32),
)


_post = pl.pallas_call(
    _post_body,
    grid=(_N // _BR,),
    in_specs=[
        pl.BlockSpec((_NSC, _BR, _F2), lambda i: (0, i, 0)),
        pl.BlockSpec((1, _F2), lambda i: (0, 0)),
        pl.BlockSpec((_F2, _H), lambda i: (0, 0)),
        pl.BlockSpec((1, _H), lambda i: (0, 0)),
        pl.BlockSpec((_H, _H), lambda i: (0, 0)),
        pl.BlockSpec((1, _H), lambda i: (0, 0)),
        pl.BlockSpec((_H, _C), lambda i: (0, 0)),
        pl.BlockSpec((1, _C), lambda i: (0, 0)),
    ],
    out_specs=pl.BlockSpec((_BR, _C), lambda i: (i, 0)),
    out_shape=jax.ShapeDtypeStruct((_N, _C), jnp.float32),
)


def _make_sc_spmm(F):
    mesh = plsc.VectorSubcoreMesh(core_axis_name="c", subcore_axis_name="s")
    nj = F // 16
    dnums = lax.GatherDimensionNumbers(
        offset_dims=(), collapsed_slice_dims=(0,), start_index_map=(0,))

    @functools.partial(
        pl.kernel,
        out_type=jax.ShapeDtypeStruct((_NSC, _N, F), jnp.float32),
        mesh=mesh,
        scratch_types=[
            pltpu.VMEM((_NCHUNK * _NB, 128), jnp.int32),
            pltpu.VMEM((_NCHUNK * _NB, 128), jnp.int32),
            pltpu.VMEM((_EPW,), jnp.float32),
            pltpu.VMEM((2, _CH, F), jnp.float32),
            pltpu.VMEM_SHARED((_N, F), jnp.float32),
            pltpu.VMEM_SHARED((_N, F), jnp.float32),
            pltpu.SemaphoreType.DMA,
            pltpu.SemaphoreType.DMA,
            pltpu.SemaphoreType.DMA,
            pltpu.SemaphoreType.DMA,
        ],
        compiler_params=pltpu.CompilerParams(use_tc_tiling_on_sc=False),
    )
    def spmm_k(g_hbm, src_hbm, dst_hbm, w_hbm, out_hbm,
               sidx_v, didx_v, wv_v, rows_v, acc, g_sp,
               gsem0, gsem1, ssem0, ssem1):
        c = lax.axis_index("c")
        s = lax.axis_index("s")
        wid = s * _NSC + c
        gsems = (gsem0, gsem1)
        ssems = (ssem0, ssem1)

        def _fire_gathers(k):
            p = k % 2
            return [
                pltpu.async_copy(
                    g_sp.at[sidx_v.at[k * _NB + b]],
                    rows_v.at[p, pl.ds(b * 128, 128)], gsems[p])
                for b in range(_NB)
            ]

        pltpu.sync_copy(g_hbm.at[pl.ds(s * _RPT, _RPT)],
                        rows_v.at[1, pl.ds(0, _RPT)])
        pltpu.sync_copy(rows_v.at[1, pl.ds(0, _RPT)],
                        g_sp.at[pl.ds(s * _RPT, _RPT)])

        nrow = _EPW // 128
        icps = [
            pltpu.async_copy(
                src_hbm.at[pl.ds(wid * nrow, nrow)], sidx_v, gsem1),
            pltpu.async_copy(
                dst_hbm.at[pl.ds(wid * nrow, nrow)], didx_v, gsem1),
            pltpu.async_copy(
                w_hbm.at[pl.ds(wid * _EPW, _EPW)], wv_v, gsem1),
        ]

        def _zbody(i, carry):
            for j in range(nj):
                rows_v[1, i, pl.ds(j * 16, 16)] = jnp.zeros(
                    (16,), jnp.float32)
            return carry
        lax.fori_loop(0, _RPT, _zbody, 0)
        for cp in icps:
            cp.wait()
        pltpu.sync_copy(rows_v.at[1, pl.ds(0, _RPT)],
                        acc.at[pl.ds(s * _RPT, _RPT)])
        plsc.subcore_barrier()
        gcps = {0: _fire_gathers(0)}

        scps = {}
        for k in range(_NCHUNK):
            p = k % 2
            if k + 1 < _NCHUNK:
                if k >= 1:
                    for cp in scps.pop(k - 1):
                        cp.wait()
                gcps[k + 1] = _fire_gathers(k + 1)
            for cp in gcps.pop(k):
                cp.wait()

            def _sbody(t, carry):
                w16 = wv_v[pl.ds(k * _CH + t * 16, 16)]
                for l in range(16):
                    wl = lax.gather(
                        w16, jnp.full((16, 1), l, jnp.int32), dnums, (1,),
                        mode=lax.GatherScatterMode.PROMISE_IN_BOUNDS)
                    e = t * 16 + l
                    for j in range(nj):
                        rows_v[p, e, pl.ds(j * 16, 16)] = (
                            rows_v[p, e, pl.ds(j * 16, 16)] * wl)
                return carry
            lax.fori_loop(0, _CH // 16, _sbody, 0)

            scps[k] = [
                pltpu.async_copy(
                    rows_v.at[p, pl.ds(b * 128, 128)],
                    acc.at[didx_v.at[k * _NB + b]], ssems[p], add=True)
                for b in range(_NB)
            ]
        for k in sorted(scps):
            for cp in scps.pop(k):
                cp.wait()
        plsc.subcore_barrier()

        pltpu.sync_copy(acc.at[pl.ds(s * _RPT, _RPT)],
                        rows_v.at[0, pl.ds(0, _RPT)])
        pltpu.sync_copy(rows_v.at[0, pl.ds(0, _RPT)],
                        out_hbm.at[c, pl.ds(s * _RPT, _RPT)])

    return spmm_k


_spmm1 = _make_sc_spmm(_F1)
_spmm2 = _make_sc_spmm(_F2)


@jax.jit
def kernel(x, edge_index, edge_weight,
           W_pre1, b_pre1, W_pre2, b_pre2,
           W_g1, b_g1, W_g2, b_g2,
           W_post1, b_post1, W_post2, b_post2,
           W_out, b_out):
    pad = _EPAD - _E
    srcp = jnp.pad(edge_index[0], (0, pad)).reshape(_EPAD // 128, 128)
    dstp = jnp.pad(edge_index[1], (0, pad)).reshape(_EPAD // 128, 128)
    wp = jnp.pad(edge_weight, (0, pad))
    inv = _invsum(wp.reshape(-1, 128))

    bf16 = jnp.bfloat16
    g = _pre(inv, x.astype(bf16), W_pre1.astype(bf16), b_pre1.reshape(1, -1),
             W_pre2.astype(bf16), b_pre2.reshape(1, -1), W_g1.astype(bf16))
    p1 = _spmm1(g, srcp, dstp, wp)
    g2 = _mid(inv, p1, b_g1.reshape(1, -1), W_g2)
    p2 = _spmm2(g2, srcp, dstp, wp)
    return _post(p2, b_g2.reshape(1, -1), W_post1.astype(bf16),
                 b_post1.reshape(1, -1), W_post2.astype(bf16),
                 b_post2.reshape(1, -1), W_out.astype(bf16),
                 b_out.reshape(1, -1))


# speedup vs baseline: 1.3639x; 1.0561x over previous
import functools

import jax
import jax.numpy as jnp
from jax import lax
from jax.experimental import pallas as pl
from jax.experimental.pallas import tpu as pltpu
from jax.experimental.pallas import tpu_sc as plsc

_N, _E, _D, _H, _C = 10000, 160000, 256, 512, 16
_F1, _F2 = 32, 16
_NSC = 2
_NSUB = 16
_NW = _NSC * _NSUB
_CH = 1024
_NB = _CH // 128
_EPAD = -(-_E // (_NW * _CH)) * (_NW * _CH)
_EPW = _EPAD // _NW
_NCHUNK = _EPW // _CH
_BR = 2000
_RPT = _N // _NSUB


def _invsum_body(w_ref, o_ref):
    o_ref[...] = (1.0 / jnp.sum(w_ref[...]))[None, None]


_invsum = pl.pallas_call(
    _invsum_body,
    out_shape=jax.ShapeDtypeStruct((1, 1), jnp.float32),
)


def _pre_body(inv_ref, x_ref, w1_ref, b1_ref, w2_ref, b2_ref, wg_ref, g_ref):
    h = jax.nn.gelu(
        (jnp.dot(x_ref[...], w1_ref[...], preferred_element_type=jnp.float32)
         + b1_ref[...]).astype(jnp.bfloat16))
    h = jax.nn.gelu(
        (jnp.dot(h, w2_ref[...], preferred_element_type=jnp.float32)
         + b2_ref[...]).astype(jnp.bfloat16))
    g_ref[...] = jnp.dot(
        h, wg_ref[...],
        preferred_element_type=jnp.float32) * inv_ref[...]


_pre = pl.pallas_call(
    _pre_body,
    grid=(_N // _BR,),
    in_specs=[
        pl.BlockSpec((1, 1), lambda i: (0, 0)),
        pl.BlockSpec((_BR, _D), lambda i: (i, 0)),
        pl.BlockSpec((_D, _H), lambda i: (0, 0)),
        pl.BlockSpec((1, _H), lambda i: (0, 0)),
        pl.BlockSpec((_H, _H), lambda i: (0, 0)),
        pl.BlockSpec((1, _H), lambda i: (0, 0)),
        pl.BlockSpec((_H, _F1), lambda i: (0, 0)),
    ],
    out_specs=pl.BlockSpec((_BR, _F1), lambda i: (i, 0)),
    out_shape=jax.ShapeDtypeStruct((_N, _F1), jnp.float32),
)


def _post_body(p_ref, b_ref, w1_ref, b1_ref, w2_ref, b2_ref, wo_ref, bo_ref,
               y_ref):
    h = jnp.maximum(p_ref[0] + p_ref[1] + b_ref[...], 0.0).astype(
        jnp.bfloat16)
    h = jax.nn.gelu(
        (jnp.dot(h, w1_ref[...], preferred_element_type=jnp.float32)
         + b1_ref[...]).astype(jnp.bfloat16))
    h = jax.nn.gelu(
        (jnp.dot(h, w2_ref[...], preferred_element_type=jnp.float32)
         + b2_ref[...]).astype(jnp.bfloat16))
    y_ref[...] = (
        jnp.dot(h, wo_ref[...], preferred_element_type=jnp.float32)
        + bo_ref[...])


def _mid_body(inv_ref, p_ref, b_ref, wg_ref, g_ref):
    h = jnp.maximum(p_ref[0] + p_ref[1] + b_ref[...], 0.0)
    g_ref[...] = jnp.dot(
        h, wg_ref[...], preferred_element_type=jnp.float32) * inv_ref[...]


_mid = pl.pallas_call(
    _mid_body,
    grid=(_N // _BR,),
    in_specs=[
        pl.BlockSpec((1, 1), lambda i: (0, 0)),
        pl.BlockSpec((_NSC, _BR, _F1), lambda i: (0, i, 0)),
        pl.BlockSpec((1, _F1), lambda i: (0, 0)),
        pl.BlockSpec((_F1, _F2), lambda i: (0, 0)),
    ],
    out_specs=pl.BlockSpec((_BR, _F2), lambda i: (i, 0)),
    out_shape=jax.ShapeDtypeStruct((_N, _F2), jnp.float32),
)


_post = pl.pallas_call(
    _post_body,
    grid=(_N // _BR,),
    in_specs=[
        pl.BlockSpec((_NSC, _BR, _F2), lambda i: (0, i, 0)),
        pl.BlockSpec((1, _F2), lambda i: (0, 0)),
        pl.BlockSpec((_F2, _H), lambda i: (0, 0)),
        pl.BlockSpec((1, _H), lambda i: (0, 0)),
        pl.BlockSpec((_H, _H), lambda i: (0, 0)),
        pl.BlockSpec((1, _H), lambda i: (0, 0)),
        pl.BlockSpec((_H, _C), lambda i: (0, 0)),
        pl.BlockSpec((1, _C), lambda i: (0, 0)),
    ],
    out_specs=pl.BlockSpec((_BR, _C), lambda i: (i, 0)),
    out_shape=jax.ShapeDtypeStruct((_N, _C), jnp.float32),
)


def _make_sc_spmm(F):
    mesh = plsc.VectorSubcoreMesh(core_axis_name="c", subcore_axis_name="s")
    nj = F // 16
    dnums = lax.GatherDimensionNumbers(
        offset_dims=(), collapsed_slice_dims=(0,), start_index_map=(0,))

    @functools.partial(
        pl.kernel,
        out_type=jax.ShapeDtypeStruct((_NSC, _N, F), jnp.float32),
        mesh=mesh,
        scratch_types=[
            pltpu.VMEM((_NCHUNK * _NB, 128), jnp.int32),
            pltpu.VMEM((_NCHUNK * _NB, 128), jnp.int32),
            pltpu.VMEM((_EPW,), jnp.float32),
            pltpu.VMEM((2, _CH, F), jnp.float32),
            pltpu.VMEM_SHARED((_N, F), jnp.float32),
            pltpu.VMEM_SHARED((_N, F), jnp.float32),
            pltpu.SemaphoreType.DMA,
            pltpu.SemaphoreType.DMA,
            pltpu.SemaphoreType.DMA,
            pltpu.SemaphoreType.DMA,
        ],
        compiler_params=pltpu.CompilerParams(use_tc_tiling_on_sc=False),
    )
    def spmm_k(g_hbm, src_hbm, dst_hbm, w_hbm, out_hbm,
               sidx_v, didx_v, wv_v, rows_v, acc, g_sp,
               gsem0, gsem1, ssem0, ssem1):
        c = lax.axis_index("c")
        s = lax.axis_index("s")
        wid = s * _NSC + c
        gsems = (gsem0, gsem1)
        ssems = (ssem0, ssem1)

        def _fire_gathers(k):
            p = k % 2
            return [
                pltpu.async_copy(
                    g_sp.at[sidx_v.at[k * _NB + b]],
                    rows_v.at[p, pl.ds(b * 128, 128)], gsems[p])
                for b in range(_NB)
            ]

        pltpu.sync_copy(g_hbm.at[pl.ds(s * _RPT, _RPT)],
                        rows_v.at[1, pl.ds(0, _RPT)])
        pltpu.sync_copy(rows_v.at[1, pl.ds(0, _RPT)],
                        g_sp.at[pl.ds(s * _RPT, _RPT)])

        nrow = _EPW // 128
        icps = [
            pltpu.async_copy(
                src_hbm.at[pl.ds(wid * nrow, nrow)], sidx_v, gsem1),
            pltpu.async_copy(
                dst_hbm.at[pl.ds(wid * nrow, nrow)], didx_v, gsem1),
            pltpu.async_copy(
                w_hbm.at[pl.ds(wid * _EPW, _EPW)], wv_v, gsem1),
        ]

        def _zbody(i, carry):
            for j in range(nj):
                rows_v[1, i, pl.ds(j * 16, 16)] = jnp.zeros(
                    (16,), jnp.float32)
            return carry
        lax.fori_loop(0, _RPT, _zbody, 0)
        for cp in icps:
            cp.wait()
        pltpu.sync_copy(rows_v.at[1, pl.ds(0, _RPT)],
                        acc.at[pl.ds(s * _RPT, _RPT)])
        plsc.subcore_barrier()
        gcps = {0: _fire_gathers(0)}

        scps = {}
        for k in range(_NCHUNK):
            p = k % 2
            if k + 1 < _NCHUNK:
                if k >= 1:
                    for cp in scps.pop(k - 1):
                        cp.wait()
                gcps[k + 1] = _fire_gathers(k + 1)
            for cp in gcps.pop(k):
                cp.wait()

            def _sbody(t, carry):
                w16 = wv_v[pl.ds(k * _CH + t * 16, 16)]
                for l in range(16):
                    wl = lax.gather(
                        w16, jnp.full((16, 1), l, jnp.int32), dnums, (1,),
                        mode=lax.GatherScatterMode.PROMISE_IN_BOUNDS)
                    e = t * 16 + l
                    for j in range(nj):
                        rows_v[p, e, pl.ds(j * 16, 16)] = (
                            rows_v[p, e, pl.ds(j * 16, 16)] * wl)
                return carry
            lax.fori_loop(0, _CH // 16, _sbody, 0)

            scps[k] = [
                pltpu.async_copy(
                    rows_v.at[p, pl.ds(b * 128, 128)],
                    acc.at[didx_v.at[k * _NB + b]], ssems[p], add=True)
                for b in range(_NB)
            ]
        for k in sorted(scps):
            for cp in scps.pop(k):
                cp.wait()
        plsc.subcore_barrier()

        pltpu.sync_copy(acc.at[pl.ds(s * _RPT, _RPT)],
                        rows_v.at[0, pl.ds(0, _RPT)])
        pltpu.sync_copy(rows_v.at[0, pl.ds(0, _RPT)],
                        out_hbm.at[c, pl.ds(s * _RPT, _RPT)])

    return spmm_k


_spmm1 = _make_sc_spmm(_F1)
_spmm2 = _make_sc_spmm(_F2)


@jax.jit
def kernel(x, edge_index, edge_weight,
           W_pre1, b_pre1, W_pre2, b_pre2,
           W_g1, b_g1, W_g2, b_g2,
           W_post1, b_post1, W_post2, b_post2,
           W_out, b_out):
    pad = _EPAD - _E
    srcp = jnp.pad(edge_index[0], (0, pad)).reshape(_EPAD // 128, 128)
    dstp = jnp.pad(edge_index[1], (0, pad)).reshape(_EPAD // 128, 128)
    wp = jnp.pad(edge_weight, (0, pad))
    inv = _invsum(wp.reshape(-1, 128))

    bf16 = jnp.bfloat16
    g = _pre(inv, x.astype(bf16), W_pre1.astype(bf16), b_pre1.reshape(1, -1),
             W_pre2.astype(bf16), b_pre2.reshape(1, -1), W_g1.astype(bf16))
    p1 = _spmm1(g, srcp, dstp, wp)
    g2 = _mid(inv, p1, b_g1.reshape(1, -1), W_g2)
    p2 = _spmm2(g2, srcp, dstp, wp)
    return _post(p2, b_g2.reshape(1, -1), W_post1.astype(bf16),
                 b_post1.reshape(1, -1), W_post2.astype(bf16),
                 b_post2.reshape(1, -1), W_out.astype(bf16),
                 b_out.reshape(1, -1))


# speedup vs baseline: 1.3807x; 1.0123x over previous
import functools

import jax
import jax.numpy as jnp
from jax import lax
from jax.experimental import pallas as pl
from jax.experimental.pallas import tpu as pltpu
from jax.experimental.pallas import tpu_sc as plsc

_N, _E, _D, _H, _C = 10000, 160000, 256, 512, 16
_F1, _F2 = 32, 16
_NSC = 2
_NSUB = 16
_NW = _NSC * _NSUB
_CH = 1024
_NB = _CH // 128
_EPAD = -(-_E // (_NW * _CH)) * (_NW * _CH)
_EPW = _EPAD // _NW
_NCHUNK = _EPW // _CH
_BR = 5000
_RPT = _N // _NSUB


def _invsum_body(w_ref, o_ref):
    o_ref[...] = (1.0 / jnp.sum(w_ref[...]))[None, None]


_invsum = pl.pallas_call(
    _invsum_body,
    out_shape=jax.ShapeDtypeStruct((1, 1), jnp.float32),
)


def _pre_body(inv_ref, x_ref, w1_ref, b1_ref, w2_ref, b2_ref, wg_ref, g_ref):
    h = jax.nn.gelu(
        (jnp.dot(x_ref[...], w1_ref[...], preferred_element_type=jnp.float32)
         + b1_ref[...]).astype(jnp.bfloat16))
    h = jax.nn.gelu(
        (jnp.dot(h, w2_ref[...], preferred_element_type=jnp.float32)
         + b2_ref[...]).astype(jnp.bfloat16))
    g_ref[...] = jnp.dot(
        h, wg_ref[...],
        preferred_element_type=jnp.float32) * inv_ref[...]


_pre = pl.pallas_call(
    _pre_body,
    grid=(_N // _BR,),
    in_specs=[
        pl.BlockSpec((1, 1), lambda i: (0, 0)),
        pl.BlockSpec((_BR, _D), lambda i: (i, 0)),
        pl.BlockSpec((_D, _H), lambda i: (0, 0)),
        pl.BlockSpec((1, _H), lambda i: (0, 0)),
        pl.BlockSpec((_H, _H), lambda i: (0, 0)),
        pl.BlockSpec((1, _H), lambda i: (0, 0)),
        pl.BlockSpec((_H, _F1), lambda i: (0, 0)),
    ],
    out_specs=pl.BlockSpec((_BR, _F1), lambda i: (i, 0)),
    out_shape=jax.ShapeDtypeStruct((_N, _F1), jnp.float32),
)


def _post_body(p_ref, b_ref, w1_ref, b1_ref, w2_ref, b2_ref, wo_ref, bo_ref,
               y_ref):
    h = jnp.maximum(p_ref[0] + p_ref[1] + b_ref[...], 0.0).astype(
        jnp.bfloat16)
    h = jax.nn.gelu(
        (jnp.dot(h, w1_ref[...], preferred_element_type=jnp.float32)
         + b1_ref[...]).astype(jnp.bfloat16))
    h = jax.nn.gelu(
        (jnp.dot(h, w2_ref[...], preferred_element_type=jnp.float32)
         + b2_ref[...]).astype(jnp.bfloat16))
    y_ref[...] = (
        jnp.dot(h, wo_ref[...], preferred_element_type=jnp.float32)
        + bo_ref[...])


def _mid_body(inv_ref, p_ref, b_ref, wg_ref, g_ref):
    h = jnp.maximum(p_ref[0] + p_ref[1] + b_ref[...], 0.0)
    g_ref[...] = jnp.dot(
        h, wg_ref[...], preferred_element_type=jnp.float32) * inv_ref[...]


_mid = pl.pallas_call(
    _mid_body,
    grid=(_N // _BR,),
    in_specs=[
        pl.BlockSpec((1, 1), lambda i: (0, 0)),
        pl.BlockSpec((_NSC, _BR, _F1), lambda i: (0, i, 0)),
        pl.BlockSpec((1, _F1), lambda i: (0, 0)),
        pl.BlockSpec((_F1, _F2), lambda i: (0, 0)),
    ],
    out_specs=pl.BlockSpec((_BR, _F2), lambda i: (i, 0)),
    out_shape=jax.ShapeDtypeStruct((_N, _F2), jnp.float32),
)


_post = pl.pallas_call(
    _post_body,
    grid=(_N // _BR,),
    in_specs=[
        pl.BlockSpec((_NSC, _BR, _F2), lambda i: (0, i, 0)),
        pl.BlockSpec((1, _F2), lambda i: (0, 0)),
        pl.BlockSpec((_F2, _H), lambda i: (0, 0)),
        pl.BlockSpec((1, _H), lambda i: (0, 0)),
        pl.BlockSpec((_H, _H), lambda i: (0, 0)),
        pl.BlockSpec((1, _H), lambda i: (0, 0)),
        pl.BlockSpec((_H, _C), lambda i: (0, 0)),
        pl.BlockSpec((1, _C), lambda i: (0, 0)),
    ],
    out_specs=pl.BlockSpec((_BR, _C), lambda i: (i, 0)),
    out_shape=jax.ShapeDtypeStruct((_N, _C), jnp.float32),
)


def _make_sc_spmm(F):
    mesh = plsc.VectorSubcoreMesh(core_axis_name="c", subcore_axis_name="s")
    nj = F // 16
    dnums = lax.GatherDimensionNumbers(
        offset_dims=(), collapsed_slice_dims=(0,), start_index_map=(0,))

    @functools.partial(
        pl.kernel,
        out_type=jax.ShapeDtypeStruct((_NSC, _N, F), jnp.float32),
        mesh=mesh,
        scratch_types=[
            pltpu.VMEM((_NCHUNK * _NB, 128), jnp.int32),
            pltpu.VMEM((_NCHUNK * _NB, 128), jnp.int32),
            pltpu.VMEM((_EPW,), jnp.float32),
            pltpu.VMEM((2, _CH, F), jnp.float32),
            pltpu.VMEM_SHARED((_N, F), jnp.float32),
            pltpu.VMEM_SHARED((_N, F), jnp.float32),
            pltpu.SemaphoreType.DMA,
            pltpu.SemaphoreType.DMA,
            pltpu.SemaphoreType.DMA,
            pltpu.SemaphoreType.DMA,
        ],
        compiler_params=pltpu.CompilerParams(use_tc_tiling_on_sc=False),
    )
    def spmm_k(g_hbm, src_hbm, dst_hbm, w_hbm, out_hbm,
               sidx_v, didx_v, wv_v, rows_v, acc, g_sp,
               gsem0, gsem1, ssem0, ssem1):
        c = lax.axis_index("c")
        s = lax.axis_index("s")
        wid = s * _NSC + c
        gsems = (gsem0, gsem1)
        ssems = (ssem0, ssem1)

        def _fire_gathers(k):
            p = k % 2
            return [
                pltpu.async_copy(
                    g_sp.at[sidx_v.at[k * _NB + b]],
                    rows_v.at[p, pl.ds(b * 128, 128)], gsems[p])
                for b in range(_NB)
            ]

        pltpu.sync_copy(g_hbm.at[pl.ds(s * _RPT, _RPT)],
                        rows_v.at[1, pl.ds(0, _RPT)])
        pltpu.sync_copy(rows_v.at[1, pl.ds(0, _RPT)],
                        g_sp.at[pl.ds(s * _RPT, _RPT)])

        nrow = _EPW // 128
        icps = [
            pltpu.async_copy(
                src_hbm.at[pl.ds(wid * nrow, nrow)], sidx_v, gsem1),
            pltpu.async_copy(
                dst_hbm.at[pl.ds(wid * nrow, nrow)], didx_v, gsem1),
            pltpu.async_copy(
                w_hbm.at[pl.ds(wid * _EPW, _EPW)], wv_v, gsem1),
        ]

        def _zbody(i, carry):
            for j in range(nj):
                rows_v[1, i, pl.ds(j * 16, 16)] = jnp.zeros(
                    (16,), jnp.float32)
            return carry
        lax.fori_loop(0, _RPT, _zbody, 0)
        for cp in icps:
            cp.wait()
        pltpu.sync_copy(rows_v.at[1, pl.ds(0, _RPT)],
                        acc.at[pl.ds(s * _RPT, _RPT)])
        plsc.subcore_barrier()
        gcps = {0: _fire_gathers(0)}

        scps = {}
        for k in range(_NCHUNK):
            p = k % 2
            if k + 1 < _NCHUNK:
                if k >= 1:
                    for cp in scps.pop(k - 1):
                        cp.wait()
                gcps[k + 1] = _fire_gathers(k + 1)
            for cp in gcps.pop(k):
                cp.wait()

            def _sbody(t, carry):
                w16 = wv_v[pl.ds(k * _CH + t * 16, 16)]
                for l in range(16):
                    wl = lax.gather(
                        w16, jnp.full((16, 1), l, jnp.int32), dnums, (1,),
                        mode=lax.GatherScatterMode.PROMISE_IN_BOUNDS)
                    e = t * 16 + l
                    for j in range(nj):
                        rows_v[p, e, pl.ds(j * 16, 16)] = (
                            rows_v[p, e, pl.ds(j * 16, 16)] * wl)
                return carry
            lax.fori_loop(0, _CH // 16, _sbody, 0)

            scps[k] = [
                pltpu.async_copy(
                    rows_v.at[p, pl.ds(b * 128, 128)],
                    acc.at[didx_v.at[k * _NB + b]], ssems[p], add=True)
                for b in range(_NB)
            ]
        for k in sorted(scps):
            for cp in scps.pop(k):
                cp.wait()
        plsc.subcore_barrier()

        pltpu.sync_copy(acc.at[pl.ds(s * _RPT, _RPT)],
                        rows_v.at[0, pl.ds(0, _RPT)])
        pltpu.sync_copy(rows_v.at[0, pl.ds(0, _RPT)],
                        out_hbm.at[c, pl.ds(s * _RPT, _RPT)])

    return spmm_k


_spmm1 = _make_sc_spmm(_F1)
_spmm2 = _make_sc_spmm(_F2)


@jax.jit
def kernel(x, edge_index, edge_weight,
           W_pre1, b_pre1, W_pre2, b_pre2,
           W_g1, b_g1, W_g2, b_g2,
           W_post1, b_post1, W_post2, b_post2,
           W_out, b_out):
    pad = _EPAD - _E
    srcp = jnp.pad(edge_index[0], (0, pad)).reshape(_EPAD // 128, 128)
    dstp = jnp.pad(edge_index[1], (0, pad)).reshape(_EPAD // 128, 128)
    wp = jnp.pad(edge_weight, (0, pad))
    inv = _invsum(wp.reshape(-1, 128))

    bf16 = jnp.bfloat16
    g = _pre(inv, x.astype(bf16), W_pre1.astype(bf16), b_pre1.reshape(1, -1),
             W_pre2.astype(bf16), b_pre2.reshape(1, -1), W_g1.astype(bf16))
    p1 = _spmm1(g, srcp, dstp, wp)
    g2 = _mid(inv, p1, b_g1.reshape(1, -1), W_g2)
    p2 = _spmm2(g2, srcp, dstp, wp)
    return _post(p2, b_g2.reshape(1, -1), W_post1.astype(bf16),
                 b_post1.reshape(1, -1), W_post2.astype(bf16),
                 b_post2.reshape(1, -1), W_out.astype(bf16),
                 b_out.reshape(1, -1))


# speedup vs baseline: 1.4010x; 1.0147x over previous
import functools

import jax
import jax.numpy as jnp
from jax import lax
from jax.experimental import pallas as pl
from jax.experimental.pallas import tpu as pltpu
from jax.experimental.pallas import tpu_sc as plsc

_N, _E, _D, _H, _C = 10000, 160000, 256, 512, 16
_F1, _F2 = 32, 16
_NSC = 2
_NSUB = 16
_NW = _NSC * _NSUB
_CH = 1024
_NB = _CH // 128
_EPAD = -(-_E // (_NW * _CH)) * (_NW * _CH)
_EPW = _EPAD // _NW
_NCHUNK = _EPW // _CH
_BR = 5000
_RPT = _N // _NSUB


def _invsum_body(w_ref, o_ref):
    o_ref[...] = (1.0 / jnp.sum(w_ref[...]))[None, None]


_invsum = pl.pallas_call(
    _invsum_body,
    out_shape=jax.ShapeDtypeStruct((1, 1), jnp.float32),
)


def _pre_body(inv_ref, x_ref, w1_ref, b1_ref, w2_ref, b2_ref, wg_ref, g_ref):
    h = jax.nn.gelu(
        (jnp.dot(x_ref[...].astype(jnp.bfloat16), w1_ref[...],
                 preferred_element_type=jnp.float32)
         + b1_ref[...]).astype(jnp.bfloat16))
    h = jax.nn.gelu(
        (jnp.dot(h, w2_ref[...], preferred_element_type=jnp.float32)
         + b2_ref[...]).astype(jnp.bfloat16))
    g_ref[...] = jnp.dot(
        h, wg_ref[...],
        preferred_element_type=jnp.float32) * inv_ref[...]


_pre = pl.pallas_call(
    _pre_body,
    grid=(_N // _BR,),
    in_specs=[
        pl.BlockSpec((1, 1), lambda i: (0, 0)),
        pl.BlockSpec((_BR, _D), lambda i: (i, 0)),
        pl.BlockSpec((_D, _H), lambda i: (0, 0)),
        pl.BlockSpec((1, _H), lambda i: (0, 0)),
        pl.BlockSpec((_H, _H), lambda i: (0, 0)),
        pl.BlockSpec((1, _H), lambda i: (0, 0)),
        pl.BlockSpec((_H, _F1), lambda i: (0, 0)),
    ],
    out_specs=pl.BlockSpec((_BR, _F1), lambda i: (i, 0)),
    out_shape=jax.ShapeDtypeStruct((_N, _F1), jnp.float32),
)


def _post_body(p_ref, b_ref, w1_ref, b1_ref, w2_ref, b2_ref, wo_ref, bo_ref,
               y_ref):
    h = jnp.maximum(p_ref[0] + p_ref[1] + b_ref[...], 0.0).astype(
        jnp.bfloat16)
    h = jax.nn.gelu(
        (jnp.dot(h, w1_ref[...], preferred_element_type=jnp.float32)
         + b1_ref[...]).astype(jnp.bfloat16))
    h = jax.nn.gelu(
        (jnp.dot(h, w2_ref[...], preferred_element_type=jnp.float32)
         + b2_ref[...]).astype(jnp.bfloat16))
    y_ref[...] = (
        jnp.dot(h, wo_ref[...], preferred_element_type=jnp.float32)
        + bo_ref[...])


def _mid_body(inv_ref, p_ref, b_ref, wg_ref, g_ref):
    h = jnp.maximum(p_ref[0] + p_ref[1] + b_ref[...], 0.0)
    g_ref[...] = jnp.dot(
        h, wg_ref[...], preferred_element_type=jnp.float32) * inv_ref[...]


_mid = pl.pallas_call(
    _mid_body,
    grid=(_N // _BR,),
    in_specs=[
        pl.BlockSpec((1, 1), lambda i: (0, 0)),
        pl.BlockSpec((_NSC, _BR, _F1), lambda i: (0, i, 0)),
        pl.BlockSpec((1, _F1), lambda i: (0, 0)),
        pl.BlockSpec((_F1, _F2), lambda i: (0, 0)),
    ],
    out_specs=pl.BlockSpec((_BR, _F2), lambda i: (i, 0)),
    out_shape=jax.ShapeDtypeStruct((_N, _F2), jnp.float32),
)


_post = pl.pallas_call(
    _post_body,
    grid=(_N // _BR,),
    in_specs=[
        pl.BlockSpec((_NSC, _BR, _F2), lambda i: (0, i, 0)),
        pl.BlockSpec((1, _F2), lambda i: (0, 0)),
        pl.BlockSpec((_F2, _H), lambda i: (0, 0)),
        pl.BlockSpec((1, _H), lambda i: (0, 0)),
        pl.BlockSpec((_H, _H), lambda i: (0, 0)),
        pl.BlockSpec((1, _H), lambda i: (0, 0)),
        pl.BlockSpec((_H, _C), lambda i: (0, 0)),
        pl.BlockSpec((1, _C), lambda i: (0, 0)),
    ],
    out_specs=pl.BlockSpec((_BR, _C), lambda i: (i, 0)),
    out_shape=jax.ShapeDtypeStruct((_N, _C), jnp.float32),
)


def _make_sc_spmm(F):
    mesh = plsc.VectorSubcoreMesh(core_axis_name="c", subcore_axis_name="s")
    nj = F // 16
    dnums = lax.GatherDimensionNumbers(
        offset_dims=(), collapsed_slice_dims=(0,), start_index_map=(0,))

    @functools.partial(
        pl.kernel,
        out_type=jax.ShapeDtypeStruct((_NSC, _N, F), jnp.float32),
        mesh=mesh,
        scratch_types=[
            pltpu.VMEM((_NCHUNK * _NB, 128), jnp.int32),
            pltpu.VMEM((_NCHUNK * _NB, 128), jnp.int32),
            pltpu.VMEM((_EPW,), jnp.float32),
            pltpu.VMEM((2, _CH, F), jnp.float32),
            pltpu.VMEM_SHARED((_N, F), jnp.float32),
            pltpu.VMEM_SHARED((_N, F), jnp.float32),
            pltpu.SemaphoreType.DMA,
            pltpu.SemaphoreType.DMA,
            pltpu.SemaphoreType.DMA,
            pltpu.SemaphoreType.DMA,
        ],
        compiler_params=pltpu.CompilerParams(use_tc_tiling_on_sc=False),
    )
    def spmm_k(g_hbm, src_hbm, dst_hbm, w_hbm, out_hbm,
               sidx_v, didx_v, wv_v, rows_v, acc, g_sp,
               gsem0, gsem1, ssem0, ssem1):
        c = lax.axis_index("c")
        s = lax.axis_index("s")
        wid = s * _NSC + c
        gsems = (gsem0, gsem1)
        ssems = (ssem0, ssem1)

        def _fire_gathers(k):
            p = k % 2
            return [
                pltpu.async_copy(
                    g_sp.at[sidx_v.at[k * _NB + b]],
                    rows_v.at[p, pl.ds(b * 128, 128)], gsems[p])
                for b in range(_NB)
            ]

        pltpu.sync_copy(g_hbm.at[pl.ds(s * _RPT, _RPT)],
                        rows_v.at[1, pl.ds(0, _RPT)])
        pltpu.sync_copy(rows_v.at[1, pl.ds(0, _RPT)],
                        g_sp.at[pl.ds(s * _RPT, _RPT)])

        nrow = _EPW // 128
        icps = [
            pltpu.async_copy(
                src_hbm.at[pl.ds(wid * nrow, nrow)], sidx_v, gsem1),
            pltpu.async_copy(
                dst_hbm.at[pl.ds(wid * nrow, nrow)], didx_v, gsem1),
            pltpu.async_copy(
                w_hbm.at[pl.ds(wid * _EPW, _EPW)], wv_v, gsem1),
        ]

        def _zbody(i, carry):
            for j in range(nj):
                rows_v[1, i, pl.ds(j * 16, 16)] = jnp.zeros(
                    (16,), jnp.float32)
            return carry
        lax.fori_loop(0, _RPT, _zbody, 0)
        for cp in icps:
            cp.wait()
        pltpu.sync_copy(rows_v.at[1, pl.ds(0, _RPT)],
                        acc.at[pl.ds(s * _RPT, _RPT)])
        plsc.subcore_barrier()
        gcps = {0: _fire_gathers(0)}

        scps = {}
        for k in range(_NCHUNK):
            p = k % 2
            if k + 1 < _NCHUNK:
                if k >= 1:
                    for cp in scps.pop(k - 1):
                        cp.wait()
                gcps[k + 1] = _fire_gathers(k + 1)
            for cp in gcps.pop(k):
                cp.wait()

            def _sbody(t, carry):
                w16 = wv_v[pl.ds(k * _CH + t * 16, 16)]
                for l in range(16):
                    wl = lax.gather(
                        w16, jnp.full((16, 1), l, jnp.int32), dnums, (1,),
                        mode=lax.GatherScatterMode.PROMISE_IN_BOUNDS)
                    e = t * 16 + l
                    for j in range(nj):
                        rows_v[p, e, pl.ds(j * 16, 16)] = (
                            rows_v[p, e, pl.ds(j * 16, 16)] * wl)
                return carry
            lax.fori_loop(0, _CH // 16, _sbody, 0)

            scps[k] = [
                pltpu.async_copy(
                    rows_v.at[p, pl.ds(b * 128, 128)],
                    acc.at[didx_v.at[k * _NB + b]], ssems[p], add=True)
                for b in range(_NB)
            ]
        for k in sorted(scps):
            for cp in scps.pop(k):
                cp.wait()
        plsc.subcore_barrier()

        pltpu.sync_copy(acc.at[pl.ds(s * _RPT, _RPT)],
                        rows_v.at[0, pl.ds(0, _RPT)])
        pltpu.sync_copy(rows_v.at[0, pl.ds(0, _RPT)],
                        out_hbm.at[c, pl.ds(s * _RPT, _RPT)])

    return spmm_k


_spmm1 = _make_sc_spmm(_F1)
_spmm2 = _make_sc_spmm(_F2)


@jax.jit
def kernel(x, edge_index, edge_weight,
           W_pre1, b_pre1, W_pre2, b_pre2,
           W_g1, b_g1, W_g2, b_g2,
           W_post1, b_post1, W_post2, b_post2,
           W_out, b_out):
    pad = _EPAD - _E
    srcp = jnp.pad(edge_index[0], (0, pad)).reshape(_EPAD // 128, 128)
    dstp = jnp.pad(edge_index[1], (0, pad)).reshape(_EPAD // 128, 128)
    wp = jnp.pad(edge_weight, (0, pad))
    inv = _invsum(wp.reshape(-1, 128))

    bf16 = jnp.bfloat16
    g = _pre(inv, x, W_pre1.astype(bf16), b_pre1.reshape(1, -1),
             W_pre2.astype(bf16), b_pre2.reshape(1, -1), W_g1.astype(bf16))
    p1 = _spmm1(g, srcp, dstp, wp)
    g2 = _mid(inv, p1, b_g1.reshape(1, -1), W_g2)
    p2 = _spmm2(g2, srcp, dstp, wp)
    return _post(p2, b_g2.reshape(1, -1), W_post1.astype(bf16),
                 b_post1.reshape(1, -1), W_post2.astype(bf16),
                 b_post2.reshape(1, -1), W_out.astype(bf16),
                 b_out.reshape(1, -1))


# speedup vs baseline: 1.4040x; 1.0021x over previous
import functools

import jax
import jax.numpy as jnp
from jax import lax
from jax.experimental import pallas as pl
from jax.experimental.pallas import tpu as pltpu
from jax.experimental.pallas import tpu_sc as plsc

_N, _E, _D, _H, _C = 10000, 160000, 256, 512, 16
_F1, _F2 = 32, 16
_NSC = 2
_NSUB = 16
_NW = _NSC * _NSUB
_CH = 1024
_NB = _CH // 128
_EPAD = -(-_E // (_NW * _CH)) * (_NW * _CH)
_EPW = _EPAD // _NW
_NCHUNK = _EPW // _CH
_BR = 5000
_RPT = _N // _NSUB


def _invsum_body(w_ref, o_ref):
    o_ref[...] = (1.0 / jnp.sum(w_ref[...]))[None, None]


_invsum = pl.pallas_call(
    _invsum_body,
    out_shape=jax.ShapeDtypeStruct((1, 1), jnp.float32),
)


def _pre_body(inv_ref, x_ref, w1_ref, b1_ref, w2_ref, b2_ref, wg_ref, g_ref):
    h = jax.nn.gelu(
        (jnp.dot(x_ref[...].astype(jnp.bfloat16), w1_ref[...],
                 preferred_element_type=jnp.float32)
         + b1_ref[...]).astype(jnp.bfloat16))
    h = jax.nn.gelu(
        (jnp.dot(h, w2_ref[...], preferred_element_type=jnp.float32)
         + b2_ref[...]).astype(jnp.bfloat16))
    g_ref[...] = jnp.dot(
        h, wg_ref[...],
        preferred_element_type=jnp.float32) * inv_ref[...]


_pre = pl.pallas_call(
    _pre_body,
    grid=(_N // _BR,),
    in_specs=[
        pl.BlockSpec((1, 1), lambda i: (0, 0)),
        pl.BlockSpec((_BR, _D), lambda i: (i, 0)),
        pl.BlockSpec((_D, _H), lambda i: (0, 0)),
        pl.BlockSpec((1, _H), lambda i: (0, 0)),
        pl.BlockSpec((_H, _H), lambda i: (0, 0)),
        pl.BlockSpec((1, _H), lambda i: (0, 0)),
        pl.BlockSpec((_H, _F1), lambda i: (0, 0)),
    ],
    out_specs=pl.BlockSpec((_BR, _F1), lambda i: (i, 0)),
    out_shape=jax.ShapeDtypeStruct((_N, _F1), jnp.float32),
)


def _post_body(p_ref, b_ref, w1_ref, b1_ref, w2_ref, b2_ref, wo_ref, bo_ref,
               y_ref):
    h = jnp.maximum(p_ref[0] + p_ref[1] + b_ref[...], 0.0).astype(
        jnp.bfloat16)
    h = jax.nn.gelu(
        (jnp.dot(h, w1_ref[...], preferred_element_type=jnp.float32)
         + b1_ref[...]).astype(jnp.bfloat16))
    h = jax.nn.gelu(
        (jnp.dot(h, w2_ref[...], preferred_element_type=jnp.float32)
         + b2_ref[...]).astype(jnp.bfloat16))
    y_ref[...] = (
        jnp.dot(h, wo_ref[...], preferred_element_type=jnp.float32)
        + bo_ref[...])


def _mid_body(inv_ref, p_ref, b_ref, wg_ref, g_ref):
    h = jnp.maximum(p_ref[0] + p_ref[1] + b_ref[...], 0.0)
    g_ref[...] = jnp.dot(
        h, wg_ref[...], preferred_element_type=jnp.float32) * inv_ref[...]


_mid = pl.pallas_call(
    _mid_body,
    grid=(_N // _BR,),
    in_specs=[
        pl.BlockSpec((1, 1), lambda i: (0, 0)),
        pl.BlockSpec((_NSC, _BR, _F1), lambda i: (0, i, 0)),
        pl.BlockSpec((1, _F1), lambda i: (0, 0)),
        pl.BlockSpec((_F1, _F2), lambda i: (0, 0)),
    ],
    out_specs=pl.BlockSpec((_BR, _F2), lambda i: (i, 0)),
    out_shape=jax.ShapeDtypeStruct((_N, _F2), jnp.float32),
)


_post = pl.pallas_call(
    _post_body,
    grid=(_N // _BR,),
    in_specs=[
        pl.BlockSpec((_NSC, _BR, _F2), lambda i: (0, i, 0)),
        pl.BlockSpec((1, _F2), lambda i: (0, 0)),
        pl.BlockSpec((_F2, _H), lambda i: (0, 0)),
        pl.BlockSpec((1, _H), lambda i: (0, 0)),
        pl.BlockSpec((_H, _H), lambda i: (0, 0)),
        pl.BlockSpec((1, _H), lambda i: (0, 0)),
        pl.BlockSpec((_H, _C), lambda i: (0, 0)),
        pl.BlockSpec((1, _C), lambda i: (0, 0)),
    ],
    out_specs=pl.BlockSpec((_BR, _C), lambda i: (i, 0)),
    out_shape=jax.ShapeDtypeStruct((_N, _C), jnp.float32),
)


def _make_sc_spmm(F):
    mesh = plsc.VectorSubcoreMesh(core_axis_name="c", subcore_axis_name="s")
    nj = F // 16
    dnums = lax.GatherDimensionNumbers(
        offset_dims=(), collapsed_slice_dims=(0,), start_index_map=(0,))

    @functools.partial(
        pl.kernel,
        out_type=jax.ShapeDtypeStruct((_NSC, _N, F), jnp.float32),
        mesh=mesh,
        scratch_types=[
            pltpu.VMEM((_NCHUNK * _NB, 128), jnp.int32),
            pltpu.VMEM((_NCHUNK * _NB, 128), jnp.int32),
            pltpu.VMEM((_EPW,), jnp.float32),
            pltpu.VMEM((2, _CH, F), jnp.float32),
            pltpu.VMEM_SHARED((_N, F), jnp.float32),
            pltpu.VMEM_SHARED((_N, F), jnp.float32),
            pltpu.SemaphoreType.DMA,
            pltpu.SemaphoreType.DMA,
            pltpu.SemaphoreType.DMA,
            pltpu.SemaphoreType.DMA,
        ],
        compiler_params=pltpu.CompilerParams(use_tc_tiling_on_sc=False),
    )
    def spmm_k(g_hbm, src_hbm, dst_hbm, w_hbm, out_hbm,
               sidx_v, didx_v, wv_v, rows_v, acc, g_sp,
               gsem0, gsem1, ssem0, ssem1):
        c = lax.axis_index("c")
        s = lax.axis_index("s")
        wid = s * _NSC + c
        gsems = (gsem0, gsem1)
        ssems = (ssem0, ssem1)

        def _fire_gathers(k):
            p = k % 2
            return [
                pltpu.async_copy(
                    g_sp.at[sidx_v.at[k * _NB + b]],
                    rows_v.at[p, pl.ds(b * 128, 128)], gsems[p])
                for b in range(_NB)
            ]

        pltpu.sync_copy(g_hbm.at[pl.ds(s * _RPT, _RPT)],
                        g_sp.at[pl.ds(s * _RPT, _RPT)])

        nrow = _EPW // 128
        icps = [
            pltpu.async_copy(
                src_hbm.at[pl.ds(wid * nrow, nrow)], sidx_v, gsem1),
            pltpu.async_copy(
                dst_hbm.at[pl.ds(wid * nrow, nrow)], didx_v, gsem1),
            pltpu.async_copy(
                w_hbm.at[pl.ds(wid * _EPW, _EPW)], wv_v, gsem1),
        ]

        def _zbody(i, carry):
            for j in range(nj):
                rows_v[1, i, pl.ds(j * 16, 16)] = jnp.zeros(
                    (16,), jnp.float32)
            return carry
        lax.fori_loop(0, _RPT, _zbody, 0)
        for cp in icps:
            cp.wait()
        pltpu.sync_copy(rows_v.at[1, pl.ds(0, _RPT)],
                        acc.at[pl.ds(s * _RPT, _RPT)])
        plsc.subcore_barrier()
        gcps = {0: _fire_gathers(0)}

        scps = {}
        for k in range(_NCHUNK):
            p = k % 2
            if k + 1 < _NCHUNK:
                if k >= 1:
                    for cp in scps.pop(k - 1):
                        cp.wait()
                gcps[k + 1] = _fire_gathers(k + 1)
            for cp in gcps.pop(k):
                cp.wait()

            def _sbody(t, carry):
                w16 = wv_v[pl.ds(k * _CH + t * 16, 16)]
                for l in range(16):
                    wl = lax.gather(
                        w16, jnp.full((16, 1), l, jnp.int32), dnums, (1,),
                        mode=lax.GatherScatterMode.PROMISE_IN_BOUNDS)
                    e = t * 16 + l
                    for j in range(nj):
                        rows_v[p, e, pl.ds(j * 16, 16)] = (
                            rows_v[p, e, pl.ds(j * 16, 16)] * wl)
                return carry
            lax.fori_loop(0, _CH // 16, _sbody, 0)

            scps[k] = [
                pltpu.async_copy(
                    rows_v.at[p, pl.ds(b * 128, 128)],
                    acc.at[didx_v.at[k * _NB + b]], ssems[p], add=True)
                for b in range(_NB)
            ]
        for k in sorted(scps):
            for cp in scps.pop(k):
                cp.wait()
        plsc.subcore_barrier()

        pltpu.sync_copy(acc.at[pl.ds(s * _RPT, _RPT)],
                        out_hbm.at[c, pl.ds(s * _RPT, _RPT)])

    return spmm_k


_spmm1 = _make_sc_spmm(_F1)
_spmm2 = _make_sc_spmm(_F2)


@jax.jit
def kernel(x, edge_index, edge_weight,
           W_pre1, b_pre1, W_pre2, b_pre2,
           W_g1, b_g1, W_g2, b_g2,
           W_post1, b_post1, W_post2, b_post2,
           W_out, b_out):
    pad = _EPAD - _E
    srcp = jnp.pad(edge_index[0], (0, pad)).reshape(_EPAD // 128, 128)
    dstp = jnp.pad(edge_index[1], (0, pad)).reshape(_EPAD // 128, 128)
    wp = jnp.pad(edge_weight, (0, pad))
    inv = _invsum(wp.reshape(-1, 128))

    bf16 = jnp.bfloat16
    g = _pre(inv, x, W_pre1.astype(bf16), b_pre1.reshape(1, -1),
             W_pre2.astype(bf16), b_pre2.reshape(1, -1), W_g1.astype(bf16))
    p1 = _spmm1(g, srcp, dstp, wp)
    g2 = _mid(inv, p1, b_g1.reshape(1, -1), W_g2)
    p2 = _spmm2(g2, srcp, dstp, wp)
    return _post(p2, b_g2.reshape(1, -1), W_post1.astype(bf16),
                 b_post1.reshape(1, -1), W_post2.astype(bf16),
                 b_post2.reshape(1, -1), W_out.astype(bf16),
                 b_out.reshape(1, -1))
